# baseline (speedup 1.0000x reference)
import functools

import jax
import jax.numpy as jnp
from jax import lax
from jax.experimental import pallas as pl
from jax.experimental.pallas import tpu as pltpu
from jax.experimental.pallas import tpu_sc as plsc

_D = 441
_B, _C, _T = 32, 2, 16384
_L = 16
_PAD = 448


def _flanger_body(x_hbm, mod_hbm, out_hbm, x0_v, x1_v, m_v, o0_v, o1_v):
    b = lax.axis_index("s") * 2 + lax.axis_index("c")
    pltpu.sync_copy(x_hbm.at[b, 0], x0_v.at[pl.ds(_PAD, _T)])
    pltpu.sync_copy(x_hbm.at[b, 1], x1_v.at[pl.ds(_PAD, _T)])
    pltpu.sync_copy(mod_hbm.at[b], m_v)

    zeros = jnp.zeros((_L,), jnp.float32)
    for j in range(_PAD // _L):
        x0_v[pl.ds(j * _L, _L)] = zeros
        x1_v[pl.ds(j * _L, _L)] = zeros

    lane = lax.iota(jnp.int32, _L)

    @plsc.parallel_loop(0, _T // _L, unroll=16)
    def body(it):
        t0 = it * _L
        tv = t0 + lane
        mv = m_v[pl.ds(t0, _L)]
        u448 = tv.astype(jnp.float32) + (jnp.float32(_PAD) - jnp.float32(_D) * mv)
        i448 = u448.astype(jnp.int32)
        frac = u448 - i448.astype(jnp.float32)
        dp = i448 - tv - _PAD
        wp = lax.shift_right_arithmetic(dp, 31)
        gp = i448 - _D - _D * wp
        wn = lax.shift_right_arithmetic(dp + 1, 31)
        gn = i448 + 1 - _D - _D * wn
        pv0 = plsc.load_gather(x0_v, [gp])
        nv0 = plsc.load_gather(x0_v, [gn])
        pv1 = plsc.load_gather(x1_v, [gp])
        nv1 = plsc.load_gather(x1_v, [gn])
        omf = 1.0 - frac
        o0_v[pl.ds(t0, _L)] = x0_v[pl.ds(t0 + _PAD, _L)] + frac * nv0 + omf * pv0
        o1_v[pl.ds(t0, _L)] = x1_v[pl.ds(t0 + _PAD, _L)] + frac * nv1 + omf * pv1

    pltpu.sync_copy(o0_v, out_hbm.at[b, 0])
    pltpu.sync_copy(o1_v, out_hbm.at[b, 1])


@jax.jit
def _flanger(x, mod_sig):
    mesh = plsc.VectorSubcoreMesh(core_axis_name="c", subcore_axis_name="s")
    fn = functools.partial(
        pl.kernel,
        mesh=mesh,
        compiler_params=pltpu.CompilerParams(
            needs_layout_passes=False, use_tc_tiling_on_sc=False
        ),
        out_type=jax.ShapeDtypeStruct((_B, _C, _T), jnp.float32),
        scratch_types=[
            pltpu.VMEM((_PAD + _T,), jnp.float32),
            pltpu.VMEM((_PAD + _T,), jnp.float32),
            pltpu.VMEM((_T,), jnp.float32),
            pltpu.VMEM((_T,), jnp.float32),
            pltpu.VMEM((_T,), jnp.float32),
        ],
    )(_flanger_body)
    return fn(x, mod_sig)


def kernel(x, mod_sig):
    return _flanger(x, mod_sig)


# speedup vs baseline: 1.2692x; 1.2692x over previous
import functools

import jax
import jax.numpy as jnp
from jax import lax
from jax.experimental import pallas as pl
from jax.experimental.pallas import tpu as pltpu
from jax.experimental.pallas import tpu_sc as plsc

_D = 441
_B, _C, _T = 32, 2, 16384
_L = 16
_PAD = 448


def _flanger_body(x_hbm, mod_hbm, out_hbm, x0_v, x1_v, m_v, o0_v, o1_v):
    b = lax.axis_index("s") * 2 + lax.axis_index("c")
    pltpu.sync_copy(x_hbm.at[b, 0], x0_v.at[pl.ds(_PAD, _T)])
    pltpu.sync_copy(x_hbm.at[b, 1], x1_v.at[pl.ds(_PAD, _T)])
    pltpu.sync_copy(mod_hbm.at[b], m_v)

    zeros = jnp.zeros((_L,), jnp.float32)
    for j in range(_PAD // _L):
        x0_v[pl.ds(j * _L, _L)] = zeros
        x1_v[pl.ds(j * _L, _L)] = zeros

    lane = lax.iota(jnp.int32, _L)

    @plsc.parallel_loop(0, _T // _L, unroll=4)
    def body(it):
        t0 = it * _L
        tv = t0 + lane
        mv = m_v[pl.ds(t0, _L)]
        u448 = tv.astype(jnp.float32) + (jnp.float32(_PAD) - jnp.float32(_D) * mv)
        i448 = u448.astype(jnp.int32)
        frac = u448 - i448.astype(jnp.float32)
        dp = i448 - tv - _PAD
        wp = lax.shift_right_arithmetic(dp, 31)
        gp = i448 - _D - _D * wp
        wn = lax.shift_right_arithmetic(dp + 1, 31)
        gn = i448 + 1 - _D - _D * wn
        pv0 = plsc.load_gather(x0_v, [gp])
        nv0 = plsc.load_gather(x0_v, [gn])
        pv1 = plsc.load_gather(x1_v, [gp])
        nv1 = plsc.load_gather(x1_v, [gn])
        omf = 1.0 - frac
        o0_v[pl.ds(t0, _L)] = x0_v[pl.ds(t0 + _PAD, _L)] + frac * nv0 + omf * pv0
        o1_v[pl.ds(t0, _L)] = x1_v[pl.ds(t0 + _PAD, _L)] + frac * nv1 + omf * pv1

    pltpu.sync_copy(o0_v, out_hbm.at[b, 0])
    pltpu.sync_copy(o1_v, out_hbm.at[b, 1])


@jax.jit
def _flanger(x, mod_sig):
    mesh = plsc.VectorSubcoreMesh(core_axis_name="c", subcore_axis_name="s")
    fn = functools.partial(
        pl.kernel,
        mesh=mesh,
        compiler_params=pltpu.CompilerParams(
            needs_layout_passes=False, use_tc_tiling_on_sc=False
        ),
        out_type=jax.ShapeDtypeStruct((_B, _C, _T), jnp.float32),
        scratch_types=[
            pltpu.VMEM((_PAD + _T,), jnp.float32),
            pltpu.VMEM((_PAD + _T,), jnp.float32),
            pltpu.VMEM((_T,), jnp.float32),
            pltpu.VMEM((_T,), jnp.float32),
            pltpu.VMEM((_T,), jnp.float32),
        ],
    )(_flanger_body)
    return fn(x, mod_sig)


def kernel(x, mod_sig):
    return _flanger(x, mod_sig)


# speedup vs baseline: 1.3647x; 1.0753x over previous
import functools

import jax
import jax.numpy as jnp
from jax import lax
from jax.experimental import pallas as pl
from jax.experimental.pallas import tpu as pltpu
from jax.experimental.pallas import tpu_sc as plsc

_D = 441
_B, _C, _T = 32, 2, 16384
_L = 16
_PAD = 448
_NCHUNK = 8
_CS = _T // _NCHUNK


def _flanger_body(x_hbm, mod_hbm, out_hbm, x0_v, x1_v, m_v, o0_v, o1_v,
                  sem_in, sem_out):
    b = lax.axis_index("s") * 2 + lax.axis_index("c")

    in_copies = []
    for c in range(_NCHUNK):
        sl = pl.ds(c * _CS, _CS)
        xsl = pl.ds(_PAD + c * _CS, _CS)
        in_copies.append((
            pltpu.async_copy(mod_hbm.at[b, sl], m_v.at[sl], sem_in),
            pltpu.async_copy(x_hbm.at[b, 0, sl], x0_v.at[xsl], sem_in),
            pltpu.async_copy(x_hbm.at[b, 1, sl], x1_v.at[xsl], sem_in),
        ))

    zeros = jnp.zeros((_L,), jnp.float32)
    for j in range(_PAD // _L):
        x0_v[pl.ds(j * _L, _L)] = zeros
        x1_v[pl.ds(j * _L, _L)] = zeros

    lane = lax.iota(jnp.int32, _L)

    out_copies = []
    for c in range(_NCHUNK):
        for cp in in_copies[c]:
            cp.wait()

        @plsc.parallel_loop(c * (_CS // _L), (c + 1) * (_CS // _L), unroll=4)
        def body(it):
            t0 = it * _L
            tv = t0 + lane
            mv = m_v[pl.ds(t0, _L)]
            u448 = tv.astype(jnp.float32) + (
                jnp.float32(_PAD) - jnp.float32(_D) * mv)
            i448 = u448.astype(jnp.int32)
            frac = u448 - i448.astype(jnp.float32)
            dp = i448 - tv - _PAD
            wp = lax.shift_right_arithmetic(dp, 31)
            gp = i448 - _D - _D * wp
            wn = lax.shift_right_arithmetic(dp + 1, 31)
            gn = i448 + 1 - _D - _D * wn
            pv0 = plsc.load_gather(x0_v, [gp])
            nv0 = plsc.load_gather(x0_v, [gn])
            pv1 = plsc.load_gather(x1_v, [gp])
            nv1 = plsc.load_gather(x1_v, [gn])
            omf = 1.0 - frac
            o0_v[pl.ds(t0, _L)] = (
                x0_v[pl.ds(t0 + _PAD, _L)] + frac * nv0 + omf * pv0)
            o1_v[pl.ds(t0, _L)] = (
                x1_v[pl.ds(t0 + _PAD, _L)] + frac * nv1 + omf * pv1)

        sl = pl.ds(c * _CS, _CS)
        out_copies.append(
            pltpu.async_copy(o0_v.at[sl], out_hbm.at[b, 0, sl], sem_out))
        out_copies.append(
            pltpu.async_copy(o1_v.at[sl], out_hbm.at[b, 1, sl], sem_out))

    for cp in out_copies:
        cp.wait()


@jax.jit
def _flanger(x, mod_sig):
    mesh = plsc.VectorSubcoreMesh(core_axis_name="c", subcore_axis_name="s")
    fn = functools.partial(
        pl.kernel,
        mesh=mesh,
        compiler_params=pltpu.CompilerParams(
            needs_layout_passes=False, use_tc_tiling_on_sc=False
        ),
        out_type=jax.ShapeDtypeStruct((_B, _C, _T), jnp.float32),
        scratch_types=[
            pltpu.VMEM((_PAD + _T,), jnp.float32),
            pltpu.VMEM((_PAD + _T,), jnp.float32),
            pltpu.VMEM((_T,), jnp.float32),
            pltpu.VMEM((_T,), jnp.float32),
            pltpu.VMEM((_T,), jnp.float32),
            pltpu.SemaphoreType.DMA,
            pltpu.SemaphoreType.DMA,
        ],
    )(_flanger_body)
    return fn(x, mod_sig)


def kernel(x, mod_sig):
    return _flanger(x, mod_sig)


# speedup vs baseline: 1.5442x; 1.1315x over previous
import functools

import jax
import jax.numpy as jnp
from jax import lax
from jax.experimental import pallas as pl
from jax.experimental.pallas import tpu as pltpu
from jax.experimental.pallas import tpu_sc as plsc

_D = 441
_B, _C, _T = 32, 2, 16384
_L = 16
_NB = _T // 128
_ROW = _C * _T
_PADB = 4
_PADW = _PADB * _C * 128
_PADT = _PADB * 128
_NCHUNK = 8
_CBLK = _NB // _NCHUNK
_CW = _CBLK * _C * 128
_CIT = _CBLK * 128 // _L


def _flanger_body(x_hbm, mod_hbm, out_hbm, x_v, m_v, o_v, idx_v,
                  sem_mod, sem_in, sem_out):
    b = lax.axis_index("s") * 2 + lax.axis_index("c")

    lane = lax.iota(jnp.int32, _L)
    mrow0 = (b // 8) * 1024 + b % 8
    lane8 = lane * 8
    for j0 in range(_NB // _L):
        idx_v[pl.ds(j0 * _L, _L)] = (mrow0 + j0 * (_L * 8)) + lane8
    mod_cp = pltpu.async_copy(mod_hbm.at[idx_v], m_v, sem_mod)

    in_cps = []
    for c in range(_NCHUNK):
        in_cps.append(pltpu.async_copy(
            x_hbm.at[b, pl.ds(c * _CW, _CW)],
            x_v.at[pl.ds(_PADW + c * _CW, _CW)], sem_in))

    zeros = jnp.zeros((_L,), jnp.float32)
    for j in range(_PADW // _L):
        x_v[pl.ds(j * _L, _L)] = zeros

    mod_cp.wait()

    out_cps = []
    for c in range(_NCHUNK):
        in_cps[c].wait()

        @plsc.parallel_loop(c * _CIT, (c + 1) * _CIT, unroll=4)
        def body(it):
            t0 = it * _L
            jblk = it // (128 // _L)
            k0 = (it % (128 // _L)) * _L
            off0 = jblk * 256 + k0
            tv = t0 + lane
            mv = m_v[jblk, pl.ds(k0, _L)]
            u = tv.astype(jnp.float32) + (
                jnp.float32(_PADT) - jnp.float32(_D) * mv)
            iu = u.astype(jnp.int32)
            frac = u - iu.astype(jnp.float32)
            dp = iu - tv - _PADT
            wp = lax.shift_right_arithmetic(dp, 31)
            gp = iu - _D - _D * wp
            wn = lax.shift_right_arithmetic(dp + 1, 31)
            gn = iu + 1 - _D - _D * wn
            ap0 = gp + gp - (gp & 127)
            an0 = gn + gn - (gn & 127)
            ap1 = ap0 + 128
            an1 = an0 + 128
            pv0 = plsc.load_gather(x_v, [ap0])
            nv0 = plsc.load_gather(x_v, [an0])
            pv1 = plsc.load_gather(x_v, [ap1])
            nv1 = plsc.load_gather(x_v, [an1])
            omf = 1.0 - frac
            x0 = x_v[pl.ds(_PADW + off0, _L)]
            x1 = x_v[pl.ds(_PADW + off0 + 128, _L)]
            o_v[pl.ds(off0, _L)] = x0 + frac * nv0 + omf * pv0
            o_v[pl.ds(off0 + 128, _L)] = x1 + frac * nv1 + omf * pv1

        out_cps.append(pltpu.async_copy(
            o_v.at[pl.ds(c * _CW, _CW)],
            out_hbm.at[b, pl.ds(c * _CW, _CW)], sem_out))

    for cp in out_cps:
        cp.wait()


@jax.jit
def _flanger(x, mod_sig):
    xl = x.reshape(_B, _C, _NB, 128).transpose(0, 2, 1, 3).reshape(_B, _ROW)
    ml = mod_sig.reshape(4, 8, _NB, 128).transpose(0, 2, 1, 3)
    ml = ml.reshape(4 * _NB * 8, 128)

    mesh = plsc.VectorSubcoreMesh(core_axis_name="c", subcore_axis_name="s")
    fn = functools.partial(
        pl.kernel,
        mesh=mesh,
        compiler_params=pltpu.CompilerParams(
            needs_layout_passes=False, use_tc_tiling_on_sc=False
        ),
        out_type=jax.ShapeDtypeStruct((_B, _ROW), jnp.float32),
        scratch_types=[
            pltpu.VMEM((_PADW + _ROW,), jnp.float32),
            pltpu.VMEM((_NB, 128), jnp.float32),
            pltpu.VMEM((_ROW,), jnp.float32),
            pltpu.VMEM((_NB,), jnp.int32),
            pltpu.SemaphoreType.DMA,
            pltpu.SemaphoreType.DMA,
            pltpu.SemaphoreType.DMA,
        ],
    )(_flanger_body)
    ol = fn(xl, ml)
    return ol.reshape(_B, _NB, _C, 128).transpose(0, 2, 1, 3).reshape(
        _B, _C, _T)


def kernel(x, mod_sig):
    return _flanger(x, mod_sig)


# speedup vs baseline: 1.6110x; 1.0433x over previous
import functools

import jax
import jax.numpy as jnp
from jax import lax
from jax.experimental import pallas as pl
from jax.experimental.pallas import tpu as pltpu
from jax.experimental.pallas import tpu_sc as plsc

_D = 441
_B, _C, _T = 32, 2, 16384
_L = 16
_NB = _T // 128
_ROW = _C * _T
_PADB = 4
_PADW = _PADB * _C * 128
_PADT = _PADB * 128
_NCHUNK = 4
_CBLK = _NB // _NCHUNK
_CW = _CBLK * _C * 128
_CIT = _CBLK * 128 // _L


def _flanger_body(x_hbm, mod_hbm, out_hbm, x_v, m_v, o_v, idx_v,
                  sem_mod, sem_in, sem_out):
    b = lax.axis_index("s") * 2 + lax.axis_index("c")

    lane = lax.iota(jnp.int32, _L)
    mrow0 = (b // 8) * 1024 + b % 8
    lane8 = lane * 8
    for j0 in range(_NB // _L):
        idx_v[pl.ds(j0 * _L, _L)] = (mrow0 + j0 * (_L * 8)) + lane8
    mod_cp = pltpu.async_copy(mod_hbm.at[idx_v], m_v, sem_mod)

    in_cps = []
    for c in range(_NCHUNK):
        in_cps.append(pltpu.async_copy(
            x_hbm.at[b, pl.ds(c * _CW, _CW)],
            x_v.at[pl.ds(_PADW + c * _CW, _CW)], sem_in))

    zeros = jnp.zeros((_L,), jnp.float32)
    for j in range(_PADW // _L):
        x_v[pl.ds(j * _L, _L)] = zeros

    mod_cp.wait()

    out_cps = []
    for c in range(_NCHUNK):
        in_cps[c].wait()

        @plsc.parallel_loop(c * _CIT, (c + 1) * _CIT, unroll=4)
        def body(it):
            t0 = it * _L
            jblk = it // (128 // _L)
            k0 = (it % (128 // _L)) * _L
            off0 = jblk * 256 + k0
            tv = t0 + lane
            mv = m_v[jblk, pl.ds(k0, _L)]
            u = tv.astype(jnp.float32) + (
                jnp.float32(_PADT) - jnp.float32(_D) * mv)
            iu = u.astype(jnp.int32)
            frac = u - iu.astype(jnp.float32)
            dp = iu - tv - _PADT
            wp = lax.shift_right_arithmetic(dp, 31)
            gp = iu - _D - _D * wp
            wn = lax.shift_right_arithmetic(dp + 1, 31)
            gn = iu + 1 - _D - _D * wn
            ap0 = gp + gp - (gp & 127)
            an0 = gn + gn - (gn & 127)
            ap1 = ap0 + 128
            an1 = an0 + 128
            pv0 = plsc.load_gather(x_v, [ap0])
            nv0 = plsc.load_gather(x_v, [an0])
            pv1 = plsc.load_gather(x_v, [ap1])
            nv1 = plsc.load_gather(x_v, [an1])
            omf = 1.0 - frac
            x0 = x_v[pl.ds(_PADW + off0, _L)]
            x1 = x_v[pl.ds(_PADW + off0 + 128, _L)]
            o_v[pl.ds(off0, _L)] = x0 + frac * nv0 + omf * pv0
            o_v[pl.ds(off0 + 128, _L)] = x1 + frac * nv1 + omf * pv1

        out_cps.append(pltpu.async_copy(
            o_v.at[pl.ds(c * _CW, _CW)],
            out_hbm.at[b, pl.ds(c * _CW, _CW)], sem_out))

    for cp in out_cps:
        cp.wait()


@jax.jit
def _flanger(x, mod_sig):
    xl = x.reshape(_B, _C, _NB, 128).transpose(0, 2, 1, 3).reshape(_B, _ROW)
    ml = mod_sig.reshape(4, 8, _NB, 128).transpose(0, 2, 1, 3)
    ml = ml.reshape(4 * _NB * 8, 128)

    mesh = plsc.VectorSubcoreMesh(core_axis_name="c", subcore_axis_name="s")
    fn = functools.partial(
        pl.kernel,
        mesh=mesh,
        compiler_params=pltpu.CompilerParams(
            needs_layout_passes=False, use_tc_tiling_on_sc=False
        ),
        out_type=jax.ShapeDtypeStruct((_B, _ROW), jnp.float32),
        scratch_types=[
            pltpu.VMEM((_PADW + _ROW,), jnp.float32),
            pltpu.VMEM((_NB, 128), jnp.float32),
            pltpu.VMEM((_ROW,), jnp.float32),
            pltpu.VMEM((_NB,), jnp.int32),
            pltpu.SemaphoreType.DMA,
            pltpu.SemaphoreType.DMA,
            pltpu.SemaphoreType.DMA,
        ],
    )(_flanger_body)
    ol = fn(xl, ml)
    return ol.reshape(_B, _NB, _C, 128).transpose(0, 2, 1, 3).reshape(
        _B, _C, _T)


def kernel(x, mod_sig):
    return _flanger(x, mod_sig)


# speedup vs baseline: 1.6342x; 1.0144x over previous
import functools

import jax
import jax.numpy as jnp
from jax import lax
from jax.experimental import pallas as pl
from jax.experimental.pallas import tpu as pltpu
from jax.experimental.pallas import tpu_sc as plsc

_D = 441
_B, _C, _T = 32, 2, 16384
_L = 16
_NB = _T // 128
_ROW = _C * _T
_PADB = 4
_PADW = _PADB * _C * 128
_PADT = _PADB * 128
_NCHUNK = 2
_CBLK = _NB // _NCHUNK
_CW = _CBLK * _C * 128
_CIT = _CBLK * 128 // _L


def _flanger_body(x_hbm, mod_hbm, out_hbm, x_v, m_v, o_v, idx_v,
                  sem_mod, sem_in, sem_out):
    b = lax.axis_index("s") * 2 + lax.axis_index("c")

    lane = lax.iota(jnp.int32, _L)
    mrow0 = (b // 8) * 1024 + b % 8
    lane8 = lane * 8
    for j0 in range(_NB // _L):
        idx_v[pl.ds(j0 * _L, _L)] = (mrow0 + j0 * (_L * 8)) + lane8
    mod_cp = pltpu.async_copy(mod_hbm.at[idx_v], m_v, sem_mod)

    in_cps = []
    for c in range(_NCHUNK):
        in_cps.append(pltpu.async_copy(
            x_hbm.at[b, pl.ds(c * _CW, _CW)],
            x_v.at[pl.ds(_PADW + c * _CW, _CW)], sem_in))

    zeros = jnp.zeros((_L,), jnp.float32)
    for j in range(_PADW // _L):
        x_v[pl.ds(j * _L, _L)] = zeros

    mod_cp.wait()

    out_cps = []
    for c in range(_NCHUNK):
        in_cps[c].wait()

        @plsc.parallel_loop(c * _CIT, (c + 1) * _CIT, unroll=4)
        def body(it):
            t0 = it * _L
            jblk = it // (128 // _L)
            k0 = (it % (128 // _L)) * _L
            off0 = jblk * 256 + k0
            tv = t0 + lane
            mv = m_v[jblk, pl.ds(k0, _L)]
            u = tv.astype(jnp.float32) + (
                jnp.float32(_PADT) - jnp.float32(_D) * mv)
            iu = u.astype(jnp.int32)
            frac = u - iu.astype(jnp.float32)
            dp = iu - tv - _PADT
            wp = lax.shift_right_arithmetic(dp, 31)
            gp = iu - _D - _D * wp
            wn = lax.shift_right_arithmetic(dp + 1, 31)
            gn = iu + 1 - _D - _D * wn
            ap0 = gp + gp - (gp & 127)
            an0 = gn + gn - (gn & 127)
            ap1 = ap0 + 128
            an1 = an0 + 128
            pv0 = plsc.load_gather(x_v, [ap0])
            nv0 = plsc.load_gather(x_v, [an0])
            pv1 = plsc.load_gather(x_v, [ap1])
            nv1 = plsc.load_gather(x_v, [an1])
            omf = 1.0 - frac
            x0 = x_v[pl.ds(_PADW + off0, _L)]
            x1 = x_v[pl.ds(_PADW + off0 + 128, _L)]
            o_v[pl.ds(off0, _L)] = x0 + frac * nv0 + omf * pv0
            o_v[pl.ds(off0 + 128, _L)] = x1 + frac * nv1 + omf * pv1

        out_cps.append(pltpu.async_copy(
            o_v.at[pl.ds(c * _CW, _CW)],
            out_hbm.at[b, pl.ds(c * _CW, _CW)], sem_out))

    for cp in out_cps:
        cp.wait()


@jax.jit
def _flanger(x, mod_sig):
    xl = x.reshape(_B, _C, _NB, 128).transpose(0, 2, 1, 3).reshape(_B, _ROW)
    ml = mod_sig.reshape(4, 8, _NB, 128).transpose(0, 2, 1, 3)
    ml = ml.reshape(4 * _NB * 8, 128)

    mesh = plsc.VectorSubcoreMesh(core_axis_name="c", subcore_axis_name="s")
    fn = functools.partial(
        pl.kernel,
        mesh=mesh,
        compiler_params=pltpu.CompilerParams(
            needs_layout_passes=False, use_tc_tiling_on_sc=False
        ),
        out_type=jax.ShapeDtypeStruct((_B, _ROW), jnp.float32),
        scratch_types=[
            pltpu.VMEM((_PADW + _ROW,), jnp.float32),
            pltpu.VMEM((_NB, 128), jnp.float32),
            pltpu.VMEM((_ROW,), jnp.float32),
            pltpu.VMEM((_NB,), jnp.int32),
            pltpu.SemaphoreType.DMA,
            pltpu.SemaphoreType.DMA,
            pltpu.SemaphoreType.DMA,
        ],
    )(_flanger_body)
    ol = fn(xl, ml)
    return ol.reshape(_B, _NB, _C, 128).transpose(0, 2, 1, 3).reshape(
        _B, _C, _T)


def kernel(x, mod_sig):
    return _flanger(x, mod_sig)
